# Initial kernel scaffold; baseline (speedup 1.0000x reference)
import jax
import jax.numpy as jnp
from jax.experimental import pallas as pl


def kernel(A_batch, feature, no_sc_idx, no_fc_idx, W_sc0, b_sc0, W_sc1, b_sc1, W_sc2, b_sc2, W_fc0, b_fc0, W_fc1, b_fc1, W_fc2, b_fc2, w1_w, w1_b, w2_w, w2_b, attention, out_w, out_b):
    raise NotImplementedError("write your pallas kernel here")


# speedup vs baseline: 1.6266x; 1.6266x over previous
import jax
import jax.numpy as jnp
from jax import lax
from jax.experimental import pallas as pl
from jax.experimental.pallas import tpu as pltpu

_B, _N, _F, _H, _OUT, _K = 32, 256, 512, 512, 8, 5
_NBR_PAD = 16


def _topk_neighbors(g, mvec):
    gsq = g * g
    nsq = jnp.sum(gsq, axis=1, keepdims=True)
    ones = jnp.ones((1, _H), jnp.float32)
    nsq_row = lax.dot_general(ones, gsq, (((1,), (1,)), ((), ())),
                              preferred_element_type=jnp.float32)
    cross = lax.dot_general(g, g, (((1,), (1,)), ((), ())),
                            preferred_element_type=jnp.float32)
    d = nsq + nsq_row - 2.0 * cross
    rows = lax.broadcasted_iota(jnp.int32, (_B, _B), 0)
    cols = lax.broadcasted_iota(jnp.int32, (_B, _B), 1)
    colsf = cols.astype(jnp.float32)
    bad = (rows == cols) | (jnp.broadcast_to(mvec, (_B, _B)) != 0)
    dmx = jnp.where(bad, 1e30 + colsf * 1e24, d)
    picks = []
    for _ in range(_K):
        mn = jnp.min(dmx, axis=1, keepdims=True)
        amf = jnp.min(jnp.where(dmx == mn, colsf, 1e9), axis=1,
                      keepdims=True)
        picks.append(amf)
        dmx = jnp.where(colsf == amf, jnp.float32(jnp.inf), dmx)
    picks.append(jnp.zeros((_B, _NBR_PAD - _K), jnp.float32))
    return jnp.concatenate(picks, axis=1)


def _gcn_body(A_ref, x_ref,
              wsc0, bsc0, wsc1, bsc1, wsc2, bsc2,
              wfc0, bfc0, wfc1, bfc1, wfc2, bfc2,
              m1_ref, m2_ref,
              es_ref, ef_ref, nbr1_ref, nbr2_ref,
              gs_ref, gf_ref):
    i = pl.program_id(0)
    A_sc = A_ref[0, 0]
    A_fc = A_ref[0, 1]
    x0 = x_ref[0]

    def layer(x, A, W, b):
        xw = jnp.dot(x, W[...], preferred_element_type=jnp.float32)
        y = jnp.dot(A, xw, preferred_element_type=jnp.float32) + b[...]
        return jnp.maximum(y, 0.0)

    xs = x0
    acc = None
    for (Wr, br) in ((wsc0, bsc0), (wsc1, bsc1), (wsc2, bsc2)):
        xs = layer(xs, A_sc, Wr, br)
        acc = xs if acc is None else acc + xs
    acc = acc * (1.0 / 3.0)
    es_ref[0] = acc
    gs_ref[pl.ds(i, 1), :] = jnp.mean(acc, axis=0, keepdims=True)

    xf = x0
    acc = None
    for (Wr, br) in ((wfc0, bfc0), (wfc1, bfc1), (wfc2, bfc2)):
        xf = layer(xf, A_fc, Wr, br)
        acc = xf if acc is None else acc + xf
    acc = acc * (1.0 / 3.0)
    ef_ref[0] = acc
    gf_ref[pl.ds(i, 1), :] = jnp.mean(acc, axis=0, keepdims=True)

    @pl.when(i == _B - 1)
    def _():
        nbr1_ref[...] = _topk_neighbors(gf_ref[...], m1_ref[...]).astype(jnp.int32)
        nbr2_ref[...] = _topk_neighbors(gs_ref[...], m2_ref[...]).astype(jnp.int32)


def _fuse_body(n1, n2, m1, m2,
               es_blk, ef_blk, w1w, w2w, w1b, w2b, att, outw, outb,
               o_ref, acc1, acc2, w1a_s, w2a_s, c1_s, c2_s):
    k = pl.program_id(1)

    @pl.when((pl.program_id(0) == 0) & (k == 0))
    def _():
        w1a_s[...] = jnp.dot(w1w[...], att[...],
                             preferred_element_type=jnp.float32)
        w2a_s[...] = jnp.dot(w2w[...], att[...],
                             preferred_element_type=jnp.float32)
        c1_s[...] = jnp.dot(w1b[...], att[...],
                            preferred_element_type=jnp.float32)
        c2_s[...] = jnp.dot(w2b[...], att[...],
                            preferred_element_type=jnp.float32)

    @pl.when(k == 0)
    def _():
        acc1[...] = es_blk[0]
        acc2[...] = ef_blk[0]

    @pl.when(k > 0)
    def _():
        acc1[...] = acc1[...] + es_blk[0]
        acc2[...] = acc2[...] + ef_blk[0]

    @pl.when(k == _K - 1)
    def _():
        e1 = acc1[...] * (1.0 / _K)
        e2 = acc2[...] * (1.0 / _K)
        s1 = jnp.dot(e1, w1a_s[...], preferred_element_type=jnp.float32) + c1_s[0, 0]
        s2 = jnp.dot(e2, w2a_s[...], preferred_element_type=jnp.float32) + c2_s[0, 0]
        s1 = jnp.where(s1 >= 0.0, s1, 0.3 * s1)
        s2 = jnp.where(s2 >= 0.0, s2, 0.3 * s2)
        mx = jnp.maximum(s1, s2)
        x1 = jnp.exp(s1 - mx)
        x2 = jnp.exp(s2 - mx)
        tot = x1 + x2
        comb = (x1 / tot) * e1 + (x2 / tot) * e2
        pooled = jnp.mean(comb, axis=0, keepdims=True)
        row = jnp.dot(pooled, outw[...],
                      preferred_element_type=jnp.float32) + outb[...]
        o_ref[pl.ds(pl.program_id(0), 1), :] = row


def kernel(A_batch, feature, no_sc_idx, no_fc_idx,
           W_sc0, b_sc0, W_sc1, b_sc1, W_sc2, b_sc2,
           W_fc0, b_fc0, W_fc1, b_fc1, W_fc2, b_fc2,
           w1_w, w1_b, w2_w, w2_b, attention, out_w, out_b):
    f32 = jnp.float32
    m1 = no_sc_idx.astype(jnp.int32).reshape(1, _B)
    m2 = no_fc_idx.astype(jnp.int32).reshape(1, _B)
    b2 = lambda b: b.reshape(1, -1).astype(f32)

    wfull = lambda shp: pl.BlockSpec(shp, lambda i: (0,) * len(shp))
    es, ef, nbr1, nbr2 = pl.pallas_call(
        _gcn_body,
        grid=(_B,),
        in_specs=[
            pl.BlockSpec((1, 2, _N, _N), lambda i: (i, 0, 0, 0)),
            pl.BlockSpec((1, _N, _F), lambda i: (i, 0, 0)),
            wfull((_F, _H)), wfull((1, _H)),
            wfull((_H, _H)), wfull((1, _H)),
            wfull((_H, _H)), wfull((1, _H)),
            wfull((_F, _H)), wfull((1, _H)),
            wfull((_H, _H)), wfull((1, _H)),
            wfull((_H, _H)), wfull((1, _H)),
            wfull((1, _B)), wfull((1, _B)),
        ],
        out_specs=[
            pl.BlockSpec((1, _N, _H), lambda i: (i, 0, 0)),
            pl.BlockSpec((1, _N, _H), lambda i: (i, 0, 0)),
            pl.BlockSpec((_B, _NBR_PAD), lambda i: (0, 0)),
            pl.BlockSpec((_B, _NBR_PAD), lambda i: (0, 0)),
        ],
        out_shape=[
            jax.ShapeDtypeStruct((_B, _N, _H), f32),
            jax.ShapeDtypeStruct((_B, _N, _H), f32),
            jax.ShapeDtypeStruct((_B, _NBR_PAD), jnp.int32),
            jax.ShapeDtypeStruct((_B, _NBR_PAD), jnp.int32),
        ],
        scratch_shapes=[
            pltpu.VMEM((_B, _H), f32),
            pltpu.VMEM((_B, _H), f32),
        ],
        compiler_params=pltpu.CompilerParams(
            dimension_semantics=("arbitrary",)),
    )(A_batch, feature,
      W_sc0, b2(b_sc0), W_sc1, b2(b_sc1), W_sc2, b2(b_sc2),
      W_fc0, b2(b_fc0), W_fc1, b2(b_fc1), W_fc2, b2(b_fc2),
      m1, m2)

    m1v = no_sc_idx.astype(jnp.int32)
    m2v = no_fc_idx.astype(jnp.int32)

    grid_spec = pltpu.PrefetchScalarGridSpec(
        num_scalar_prefetch=4,
        grid=(_B, _K),
        in_specs=[
            pl.BlockSpec(
                (1, _N, _H),
                lambda b, k, n1, n2, mm1, mm2:
                    (jnp.where(mm1[b] != 0, n1[b, k], b), 0, 0)),
            pl.BlockSpec(
                (1, _N, _H),
                lambda b, k, n1, n2, mm1, mm2:
                    (jnp.where(mm2[b] != 0, n2[b, k], b), 0, 0)),
            pl.BlockSpec((_H, _H), lambda *_: (0, 0)),
            pl.BlockSpec((_H, _H), lambda *_: (0, 0)),
            pl.BlockSpec((1, _H), lambda *_: (0, 0)),
            pl.BlockSpec((1, _H), lambda *_: (0, 0)),
            pl.BlockSpec((_H, 1), lambda *_: (0, 0)),
            pl.BlockSpec((_H, _OUT), lambda *_: (0, 0)),
            pl.BlockSpec((1, _OUT), lambda *_: (0, 0)),
        ],
        out_specs=pl.BlockSpec((_B, _OUT), lambda b, k, *_: (0, 0)),
        scratch_shapes=[
            pltpu.VMEM((_N, _H), f32),
            pltpu.VMEM((_N, _H), f32),
            pltpu.VMEM((_H, 1), f32),
            pltpu.VMEM((_H, 1), f32),
            pltpu.VMEM((1, 1), f32),
            pltpu.VMEM((1, 1), f32),
        ],
    )
    out = pl.pallas_call(
        _fuse_body,
        grid_spec=grid_spec,
        out_shape=jax.ShapeDtypeStruct((_B, _OUT), f32),
        compiler_params=pltpu.CompilerParams(
            dimension_semantics=("arbitrary", "arbitrary")),
    )(nbr1, nbr2, m1v, m2v,
      es, ef, w1_w, w2_w, b2(w1_b), b2(w2_b), attention, out_w, b2(out_b))
    return out
